# baseline (speedup 1.0000x reference)
import functools

import jax
import jax.numpy as jnp
from jax import lax
from jax.experimental import pallas as pl
from jax.experimental.pallas import tpu as pltpu
from jax.experimental.pallas import tpu_sc as plsc

N = 10000
D = 128
E = 320000
NREL = 5
W = 144
NPAD = 10240
NC, NS = 2, 16
NW = NC * NS
ROWS_PER_TILE = NPAD // NS
HALF = ROWS_PER_TILE // 2
NCH = 114
CHUNK = 88
EPT = NCH * CHUNK
NQ = 3
QCH = NCH // NQ
ZROWS = 20

_f32 = jnp.float32


RB1 = 1280


def _table_body(x_ref, w_ref, o_ref):
    y = jnp.dot(x_ref[0], w_ref[0], preferred_element_type=_f32)
    o_ref[0, :, pl.ds(0, D)] = y
    tail = jnp.concatenate(
        [jnp.ones((RB1, 1), _f32), jnp.zeros((RB1, W - D - 1), _f32)], axis=1)
    o_ref[0, :, pl.ds(D, W - D)] = tail


def _build_tables(xs, wlT):
    return pl.pallas_call(
        _table_body,
        grid=(NREL, NPAD // RB1),
        in_specs=[
            pl.BlockSpec((1, RB1, D), lambda r, i: (r, i, 0)),
            pl.BlockSpec((1, D, D), lambda r, i: (r, 0, 0)),
        ],
        out_specs=pl.BlockSpec((1, RB1, W), lambda r, i: (r, i, 0)),
        out_shape=jax.ShapeDtypeStruct((NREL, NPAD, W), _f32),
    )(xs, wlT)


def _segsum_body(table_hbm, src_hbm, dst_hbm, out_hbm,
                 src_v, dst_v, bufs, zbuf, accum, gsems, ssems, sem):
    cid = lax.axis_index("c")
    sid = lax.axis_index("s")
    wid = cid * NS + sid
    row0 = sid * ROWS_PER_TILE

    def gather(c, b):
        pltpu.async_copy(table_hbm.at[src_v.at[c]], bufs.at[b],
                         gsems.at[b])

    def wait_gather(b):
        pltpu.make_async_copy(table_hbm.at[src_v.at[0]], bufs.at[b],
                              gsems.at[b]).wait()

    def scatter(c, b):
        pltpu.async_copy(bufs.at[b], accum.at[dst_v.at[c]], ssems.at[b],
                         add=True)

    def wait_scatter(b):
        pltpu.make_async_copy(bufs.at[b], accum.at[dst_v.at[0]],
                              ssems.at[b]).wait()

    zv = jnp.zeros((16,), _f32)

    def zrow(i, carry):
        for j in range(W // 16):
            zbuf[i, pl.ds(j * 16, 16)] = zv
        return carry

    lax.fori_loop(0, ZROWS, zrow, 0)

    def zero_slice():
        def zcopy(k, carry):
            pltpu.sync_copy(zbuf, accum.at[pl.ds(row0 + k * ZROWS, ZROWS)])
            return carry
        lax.fori_loop(0, ROWS_PER_TILE // ZROWS, zcopy, 0)

    zero_slice()

    for r in range(NREL):
        plsc.subcore_barrier()
        for q in range(NQ):
            pltpu.sync_copy(src_hbm.at[r, wid, pl.ds(q * QCH, QCH)], src_v)
            pltpu.sync_copy(dst_hbm.at[r, wid, pl.ds(q * QCH, QCH)], dst_v)

            gather(0, 0)

            def step(p, carry):
                c0 = p * 2
                wait_gather(0)
                gather(c0 + 1, 1)
                scatter(c0, 0)
                wait_gather(1)
                scatter(c0 + 1, 1)
                wait_scatter(0)

                @pl.when(p < QCH // 2 - 1)
                def _():
                    gather(c0 + 2, 0)

                wait_scatter(1)
                return carry

            lax.fori_loop(0, QCH // 2, step, 0)
        plsc.subcore_barrier()

        pltpu.sync_copy(accum.at[pl.ds(row0, ROWS_PER_TILE)],
                        out_hbm.at[r, cid, pl.ds(row0, ROWS_PER_TILE)])
        if r < NREL - 1:
            zero_slice()


_segsum = functools.partial(
    pl.kernel,
    out_type=jax.ShapeDtypeStruct((NREL, NC, NPAD, W), _f32),
    mesh=plsc.VectorSubcoreMesh(core_axis_name="c", subcore_axis_name="s"),
    scratch_types=[
        pltpu.VMEM((QCH, CHUNK), jnp.int32),
        pltpu.VMEM((QCH, CHUNK), jnp.int32),
        pltpu.VMEM((2, CHUNK, W), _f32),
        pltpu.VMEM((ZROWS, W), _f32),
        pltpu.VMEM_SHARED((NPAD, W), _f32),
        pltpu.SemaphoreType.DMA((4,)),
        pltpu.SemaphoreType.DMA((4,)),
        pltpu.SemaphoreType.DMA,
    ],
    compiler_params=pltpu.CompilerParams(use_tc_tiling_on_sc=False),
)(_segsum_body)


RB3 = 1000


def _combine_body(p_ref, xj_ref, wr_ref, bl_ref, o_ref):
    acc = jnp.dot(xj_ref[...], wr_ref[...], preferred_element_type=_f32)
    acc = acc + bl_ref[...]
    for r in range(NREL):
        s = p_ref[r, 0, :, pl.ds(0, D)] + p_ref[r, 1, :, pl.ds(0, D)]
        c = p_ref[r, 0, :, pl.ds(D, 1)] + p_ref[r, 1, :, pl.ds(D, 1)]
        acc = acc + s / jnp.maximum(c, 1.0)
    o_ref[...] = jnp.maximum(acc, 0.0)


def _combine(partial, x_job, wrT, bl):
    return pl.pallas_call(
        _combine_body,
        grid=(N // RB3,),
        in_specs=[
            pl.BlockSpec((NREL, NC, RB3, W), lambda i: (0, 0, i, 0)),
            pl.BlockSpec((RB3, D), lambda i: (i, 0)),
            pl.BlockSpec((D, D), lambda i: (0, 0)),
            pl.BlockSpec((1, D), lambda i: (0, 0)),
        ],
        out_specs=pl.BlockSpec((RB3, D), lambda i: (i, 0)),
        out_shape=jax.ShapeDtypeStruct((N, D), _f32),
    )(partial, x_job, wrT, bl)


def _prep_edges(ei):
    src = ei[0].astype(jnp.int32)
    dst = ei[1].astype(jnp.int32)
    src = jnp.pad(src, (0, NW * EPT - E))
    dst = jnp.pad(dst, (0, NW * EPT - E), constant_values=N)
    return src.reshape(NW, NCH, CHUNK), dst.reshape(NW, NCH, CHUNK)


def kernel(x_station, x_machine, x_robot, x_job,
           edge_can_load, edge_loaded, edge_will_execute, edge_execute,
           edge_hold,
           Wl_can_load, bl_can_load, Wr_can_load,
           Wl_loaded, bl_loaded, Wr_loaded,
           Wl_will_execute, bl_will_execute, Wr_will_execute,
           Wl_execute, bl_execute, Wr_execute,
           Wl_hold, bl_hold, Wr_hold):
    xs = jnp.stack([x_station, x_station, x_machine, x_machine, x_robot])
    xs = jnp.pad(xs, ((0, 0), (0, NPAD - N), (0, 0)))
    wlT = jnp.stack([Wl_can_load.T, Wl_loaded.T, Wl_will_execute.T,
                     Wl_execute.T, Wl_hold.T])
    table = _build_tables(xs, wlT).reshape(NREL * NPAD, W)

    pairs = [_prep_edges(e) for e in (edge_can_load, edge_loaded,
                                      edge_will_execute, edge_execute,
                                      edge_hold)]
    src_all = jnp.stack([p[0] for p in pairs])
    src_all = src_all + (jnp.arange(NREL, dtype=jnp.int32)
                         * NPAD)[:, None, None, None]
    dst_all = jnp.stack([p[1] for p in pairs])

    partial = _segsum(table, src_all, dst_all)

    wrT = (Wr_can_load + Wr_loaded + Wr_will_execute + Wr_execute + Wr_hold).T
    bl = (bl_can_load + bl_loaded + bl_will_execute + bl_execute
          + bl_hold).reshape(1, D)
    return _combine(partial, x_job, wrT, bl)


# speedup vs baseline: 1.2150x; 1.2150x over previous
import functools

import jax
import jax.numpy as jnp
from jax import lax
from jax.experimental import pallas as pl
from jax.experimental.pallas import tpu as pltpu
from jax.experimental.pallas import tpu_sc as plsc

N = 10000
D = 128
E = 320000
NREL = 5
W = 144
NPAD = 10240
NC, NS = 2, 16
NW = NC * NS
ROWS_PER_TILE = NPAD // NS
HALF = ROWS_PER_TILE // 2
NCH = 125
CHUNK = 80
EPT = NCH * CHUNK
NQ = 5
QCH = NCH // NQ
ZROWS = 20

_f32 = jnp.float32


RB1 = 1280


def _table_body(x_ref, w_ref, o_ref):
    y = jnp.dot(x_ref[0], w_ref[0], preferred_element_type=_f32)
    o_ref[0, :, pl.ds(0, D)] = y
    tail = jnp.concatenate(
        [jnp.ones((RB1, 1), _f32), jnp.zeros((RB1, W - D - 1), _f32)], axis=1)
    o_ref[0, :, pl.ds(D, W - D)] = tail


def _build_tables(xs, wlT):
    return pl.pallas_call(
        _table_body,
        grid=(NREL, NPAD // RB1),
        in_specs=[
            pl.BlockSpec((1, RB1, D), lambda r, i: (r, i, 0)),
            pl.BlockSpec((1, D, D), lambda r, i: (r, 0, 0)),
        ],
        out_specs=pl.BlockSpec((1, RB1, W), lambda r, i: (r, i, 0)),
        out_shape=jax.ShapeDtypeStruct((NREL, NPAD, W), _f32),
    )(xs, wlT)


def _segsum_body(table_hbm,
                 src0, dst0, src1, dst1, src2, dst2, src3, dst3, src4, dst4,
                 out_hbm,
                 src_v, dst_v, bufs, zbuf, accum, gsems, ssems, sem):
    cid = lax.axis_index("c")
    sid = lax.axis_index("s")
    wid = cid * NS + sid
    row0 = sid * ROWS_PER_TILE
    edge_refs = [(src0, dst0), (src1, dst1), (src2, dst2), (src3, dst3),
                 (src4, dst4)]

    def gather(r, c, b):
        pltpu.async_copy(table_hbm.at[r].at[src_v.at[c]], bufs.at[b],
                         gsems.at[b])

    def wait_gather(b):
        pltpu.make_async_copy(table_hbm.at[0].at[src_v.at[0]], bufs.at[b],
                              gsems.at[b]).wait()

    def scatter(c, b):
        pltpu.async_copy(bufs.at[b], accum.at[dst_v.at[c]], ssems.at[b],
                         add=True)

    def wait_scatter(b):
        pltpu.make_async_copy(bufs.at[b], accum.at[dst_v.at[0]],
                              ssems.at[b]).wait()

    zv = jnp.zeros((16,), _f32)

    def zrow(i, carry):
        for j in range(W // 16):
            zbuf[i, pl.ds(j * 16, 16)] = zv
        return carry

    lax.fori_loop(0, ZROWS, zrow, 0)

    def zero_slice():
        def zcopy(k, carry):
            pltpu.sync_copy(zbuf, accum.at[pl.ds(row0 + k * ZROWS, ZROWS)])
            return carry
        lax.fori_loop(0, ROWS_PER_TILE // ZROWS, zcopy, 0)

    zero_slice()

    for r in range(NREL):
        plsc.subcore_barrier()
        src_hbm, dst_hbm = edge_refs[r]
        for q in range(NQ):
            pltpu.sync_copy(src_hbm.at[wid, pl.ds(q * QCH, QCH)], src_v)
            pltpu.sync_copy(dst_hbm.at[wid, pl.ds(q * QCH, QCH)], dst_v)

            gather(r, 0, 0)

            def step(p, carry):
                c0 = p * 2
                wait_gather(0)
                gather(r, c0 + 1, 1)
                scatter(c0, 0)
                wait_gather(1)
                scatter(c0 + 1, 1)
                wait_scatter(0)
                gather(r, c0 + 2, 0)
                wait_scatter(1)
                return carry

            lax.fori_loop(0, QCH // 2, step, 0)
            wait_gather(0)
            scatter(QCH - 1, 0)
            wait_scatter(0)
        plsc.subcore_barrier()

        pltpu.sync_copy(accum.at[pl.ds(row0, ROWS_PER_TILE)],
                        out_hbm.at[r, cid, pl.ds(row0, ROWS_PER_TILE)])
        if r < NREL - 1:
            zero_slice()


_segsum = functools.partial(
    pl.kernel,
    out_type=jax.ShapeDtypeStruct((NREL, NC, NPAD, W), _f32),
    mesh=plsc.VectorSubcoreMesh(core_axis_name="c", subcore_axis_name="s"),
    scratch_types=[
        pltpu.VMEM((QCH, CHUNK), jnp.int32),
        pltpu.VMEM((QCH, CHUNK), jnp.int32),
        pltpu.VMEM((2, CHUNK, W), _f32),
        pltpu.VMEM((ZROWS, W), _f32),
        pltpu.VMEM_SHARED((NPAD, W), _f32),
        pltpu.SemaphoreType.DMA((4,)),
        pltpu.SemaphoreType.DMA((4,)),
        pltpu.SemaphoreType.DMA,
    ],
    compiler_params=pltpu.CompilerParams(use_tc_tiling_on_sc=False),
)(_segsum_body)


RB3 = 1000


def _combine_body(p_ref, xj_ref, wr_ref, bl_ref, o_ref):
    acc = jnp.dot(xj_ref[...], wr_ref[...], preferred_element_type=_f32)
    acc = acc + bl_ref[...]
    for r in range(NREL):
        s = p_ref[r, 0, :, pl.ds(0, D)] + p_ref[r, 1, :, pl.ds(0, D)]
        c = p_ref[r, 0, :, pl.ds(D, 1)] + p_ref[r, 1, :, pl.ds(D, 1)]
        acc = acc + s / jnp.maximum(c, 1.0)
    o_ref[...] = jnp.maximum(acc, 0.0)


def _combine(partial, x_job, wrT, bl):
    return pl.pallas_call(
        _combine_body,
        grid=(N // RB3,),
        in_specs=[
            pl.BlockSpec((NREL, NC, RB3, W), lambda i: (0, 0, i, 0)),
            pl.BlockSpec((RB3, D), lambda i: (i, 0)),
            pl.BlockSpec((D, D), lambda i: (0, 0)),
            pl.BlockSpec((1, D), lambda i: (0, 0)),
        ],
        out_specs=pl.BlockSpec((RB3, D), lambda i: (i, 0)),
        out_shape=jax.ShapeDtypeStruct((N, D), _f32),
    )(partial, x_job, wrT, bl)


def _prep_edges(ei):
    ei = ei.astype(jnp.int32)
    return ei[0].reshape(NW, NCH, CHUNK), ei[1].reshape(NW, NCH, CHUNK)


def kernel(x_station, x_machine, x_robot, x_job,
           edge_can_load, edge_loaded, edge_will_execute, edge_execute,
           edge_hold,
           Wl_can_load, bl_can_load, Wr_can_load,
           Wl_loaded, bl_loaded, Wr_loaded,
           Wl_will_execute, bl_will_execute, Wr_will_execute,
           Wl_execute, bl_execute, Wr_execute,
           Wl_hold, bl_hold, Wr_hold):
    xs = jnp.stack([x_station, x_station, x_machine, x_machine, x_robot])
    xs = jnp.pad(xs, ((0, 0), (0, NPAD - N), (0, 0)))
    wlT = jnp.stack([Wl_can_load.T, Wl_loaded.T, Wl_will_execute.T,
                     Wl_execute.T, Wl_hold.T])
    table = _build_tables(xs, wlT)

    pairs = [_prep_edges(e) for e in (edge_can_load, edge_loaded,
                                      edge_will_execute, edge_execute,
                                      edge_hold)]
    flat = [a for p in pairs for a in p]
    partial = _segsum(table, *flat)

    wrT = (Wr_can_load + Wr_loaded + Wr_will_execute + Wr_execute + Wr_hold).T
    bl = (bl_can_load + bl_loaded + bl_will_execute + bl_execute
          + bl_hold).reshape(1, D)
    return _combine(partial, x_job, wrT, bl)


# speedup vs baseline: 1.5799x; 1.3004x over previous
import functools

import jax
import jax.numpy as jnp
from jax import lax
from jax.experimental import pallas as pl
from jax.experimental.pallas import tpu as pltpu
from jax.experimental.pallas import tpu_sc as plsc

N = 10000
D = 128
E = 320000
NREL = 5
W = 144
NPAD = 10240
NC, NS = 2, 16
NW = NC * NS
ROWS_PER_TILE = NPAD // NS
HALF = ROWS_PER_TILE // 2
NCH = 125
CHUNK = 80
EPT = NCH * CHUNK
NQ = 5
QCH = NCH // NQ

_f32 = jnp.float32


RB1 = 1280


def _table_body(x_ref, w_ref, o_ref):
    y = jnp.dot(x_ref[0], w_ref[0], preferred_element_type=_f32)
    o_ref[0, :, pl.ds(0, D)] = y
    tail = jnp.concatenate(
        [jnp.ones((RB1, 1), _f32), jnp.zeros((RB1, W - D - 1), _f32)], axis=1)
    o_ref[0, :, pl.ds(D, W - D)] = tail


def _build_tables(xs, wlT):
    return pl.pallas_call(
        _table_body,
        grid=(NREL, NPAD // RB1),
        in_specs=[
            pl.BlockSpec((1, RB1, D), lambda r, i: (r, i, 0)),
            pl.BlockSpec((1, D, D), lambda r, i: (r, 0, 0)),
        ],
        out_specs=pl.BlockSpec((1, RB1, W), lambda r, i: (r, i, 0)),
        out_shape=jax.ShapeDtypeStruct((NREL, NPAD, W), _f32),
    )(xs, wlT)


def _segsum_body(table_hbm,
                 src0, dst0, src1, dst1, src2, dst2, src3, dst3, src4, dst4,
                 out_hbm,
                 src_v, dst_v, bufs, accum, gsems, ssems, sem):
    cid = lax.axis_index("c")
    sid = lax.axis_index("s")
    wid = cid * NS + sid
    row0 = sid * ROWS_PER_TILE
    edge_refs = [(src0, dst0), (src1, dst1), (src2, dst2), (src3, dst3),
                 (src4, dst4)]

    def gather(r, c, b):
        pltpu.async_copy(table_hbm.at[r].at[src_v.at[c]], bufs.at[b],
                         gsems.at[b])

    def wait_gather(b):
        pltpu.make_async_copy(table_hbm.at[0].at[src_v.at[0]], bufs.at[b],
                              gsems.at[b]).wait()

    def scatter(c, b):
        pltpu.async_copy(bufs.at[b], accum.at[dst_v.at[c]], ssems.at[b],
                         add=True)

    def wait_scatter(b):
        pltpu.make_async_copy(bufs.at[b], accum.at[dst_v.at[0]],
                              ssems.at[b]).wait()

    zv = jnp.zeros((16,), _f32)

    def zrow(i, carry):
        for j in range(W // 16):
            bufs[0, i, pl.ds(j * 16, 16)] = zv
        return carry

    def zero_slice():
        lax.fori_loop(0, CHUNK, zrow, 0)

        def zcopy(k, carry):
            pltpu.sync_copy(bufs.at[0],
                            accum.at[pl.ds(row0 + k * CHUNK, CHUNK)])
            return carry
        lax.fori_loop(0, ROWS_PER_TILE // CHUNK, zcopy, 0)

    zero_slice()

    for r in range(NREL):
        plsc.subcore_barrier()
        src_hbm, dst_hbm = edge_refs[r]
        for q in range(NQ):
            pltpu.sync_copy(src_hbm.at[wid, pl.ds(q * QCH, QCH)], src_v)
            pltpu.sync_copy(dst_hbm.at[wid, pl.ds(q * QCH, QCH)], dst_v)

            gather(r, 0, 0)
            gather(r, 1, 1)

            def step(p, carry):
                c0 = p * 3
                wait_gather(0)
                gather(r, c0 + 2, 2)
                pltpu.sync_copy(bufs.at[0], accum.at[dst_v.at[c0]],
                                add=True)
                wait_gather(1)
                gather(r, c0 + 3, 0)
                pltpu.sync_copy(bufs.at[1], accum.at[dst_v.at[c0 + 1]],
                                add=True)
                wait_gather(2)

                @pl.when(c0 + 4 < QCH)
                def _():
                    gather(r, c0 + 4, 1)

                pltpu.sync_copy(bufs.at[2], accum.at[dst_v.at[c0 + 2]],
                                add=True)
                return carry

            lax.fori_loop(0, QCH // 3, step, 0)
            wait_gather(0)
            pltpu.sync_copy(bufs.at[0], accum.at[dst_v.at[QCH - 1]],
                            add=True)
        plsc.subcore_barrier()

        pltpu.sync_copy(accum.at[pl.ds(row0, ROWS_PER_TILE)],
                        out_hbm.at[r, cid, pl.ds(row0, ROWS_PER_TILE)])
        if r < NREL - 1:
            zero_slice()


_segsum = functools.partial(
    pl.kernel,
    out_type=jax.ShapeDtypeStruct((NREL, NC, NPAD, W), _f32),
    mesh=plsc.VectorSubcoreMesh(core_axis_name="c", subcore_axis_name="s"),
    scratch_types=[
        pltpu.VMEM((QCH, CHUNK), jnp.int32),
        pltpu.VMEM((QCH, CHUNK), jnp.int32),
        pltpu.VMEM((3, CHUNK, W), _f32),
        pltpu.VMEM_SHARED((NPAD, W), _f32),
        pltpu.SemaphoreType.DMA((4,)),
        pltpu.SemaphoreType.DMA((4,)),
        pltpu.SemaphoreType.DMA,
    ],
    compiler_params=pltpu.CompilerParams(use_tc_tiling_on_sc=False),
)(_segsum_body)


RB3 = 1000


def _combine_body(p_ref, xj_ref, wr_ref, bl_ref, o_ref):
    acc = jnp.dot(xj_ref[...], wr_ref[...], preferred_element_type=_f32)
    acc = acc + bl_ref[...]
    for r in range(NREL):
        s = p_ref[r, 0, :, pl.ds(0, D)] + p_ref[r, 1, :, pl.ds(0, D)]
        c = p_ref[r, 0, :, pl.ds(D, 1)] + p_ref[r, 1, :, pl.ds(D, 1)]
        acc = acc + s / jnp.maximum(c, 1.0)
    o_ref[...] = jnp.maximum(acc, 0.0)


def _combine(partial, x_job, wrT, bl):
    return pl.pallas_call(
        _combine_body,
        grid=(N // RB3,),
        in_specs=[
            pl.BlockSpec((NREL, NC, RB3, W), lambda i: (0, 0, i, 0)),
            pl.BlockSpec((RB3, D), lambda i: (i, 0)),
            pl.BlockSpec((D, D), lambda i: (0, 0)),
            pl.BlockSpec((1, D), lambda i: (0, 0)),
        ],
        out_specs=pl.BlockSpec((RB3, D), lambda i: (i, 0)),
        out_shape=jax.ShapeDtypeStruct((N, D), _f32),
    )(partial, x_job, wrT, bl)


def _prep_edges(ei):
    ei = ei.astype(jnp.int32)
    return ei[0].reshape(NW, NCH, CHUNK), ei[1].reshape(NW, NCH, CHUNK)


def kernel(x_station, x_machine, x_robot, x_job,
           edge_can_load, edge_loaded, edge_will_execute, edge_execute,
           edge_hold,
           Wl_can_load, bl_can_load, Wr_can_load,
           Wl_loaded, bl_loaded, Wr_loaded,
           Wl_will_execute, bl_will_execute, Wr_will_execute,
           Wl_execute, bl_execute, Wr_execute,
           Wl_hold, bl_hold, Wr_hold):
    xs = jnp.stack([x_station, x_station, x_machine, x_machine, x_robot])
    xs = jnp.pad(xs, ((0, 0), (0, NPAD - N), (0, 0)))
    wlT = jnp.stack([Wl_can_load.T, Wl_loaded.T, Wl_will_execute.T,
                     Wl_execute.T, Wl_hold.T])
    table = _build_tables(xs, wlT)

    pairs = [_prep_edges(e) for e in (edge_can_load, edge_loaded,
                                      edge_will_execute, edge_execute,
                                      edge_hold)]
    flat = [a for p in pairs for a in p]
    partial = _segsum(table, *flat)

    wrT = (Wr_can_load + Wr_loaded + Wr_will_execute + Wr_execute + Wr_hold).T
    bl = (bl_can_load + bl_loaded + bl_will_execute + bl_execute
          + bl_hold).reshape(1, D)
    return _combine(partial, x_job, wrT, bl)


# speedup vs baseline: 1.6793x; 1.0629x over previous
import functools

import jax
import jax.numpy as jnp
from jax import lax
from jax.experimental import pallas as pl
from jax.experimental.pallas import tpu as pltpu
from jax.experimental.pallas import tpu_sc as plsc

N = 10000
D = 128
E = 320000
NREL = 5
W = 144
NPAD = 10240
NC, NS = 2, 16
NW = NC * NS
ROWS_PER_TILE = NPAD // NS
HALF = ROWS_PER_TILE // 2
NCH = 125
CHUNK = 80
EPT = NCH * CHUNK
NQ = 5
QCH = NCH // NQ

_f32 = jnp.float32


RB1 = 1280


def _table_body(xs_ref, xm_ref, xr_ref, w_ref, o_ref):
    tail = jnp.concatenate(
        [jnp.ones((RB1, 1), _f32), jnp.zeros((RB1, W - D - 1), _f32)], axis=1)
    srcs = (xs_ref, xs_ref, xm_ref, xm_ref, xr_ref)
    for r in range(NREL):
        y = jnp.dot(srcs[r][...], w_ref[r], preferred_element_type=_f32)
        o_ref[r, :, pl.ds(0, D)] = y
        o_ref[r, :, pl.ds(D, W - D)] = tail


def _build_tables(x_station, x_machine, x_robot, wlT):
    xspec = pl.BlockSpec((RB1, D), lambda i: (i, 0))
    return pl.pallas_call(
        _table_body,
        grid=(NPAD // RB1,),
        in_specs=[
            xspec, xspec, xspec,
            pl.BlockSpec((NREL, D, D), lambda i: (0, 0, 0)),
        ],
        out_specs=pl.BlockSpec((NREL, RB1, W), lambda i: (0, i, 0)),
        out_shape=jax.ShapeDtypeStruct((NREL, NPAD, W), _f32),
    )(x_station, x_machine, x_robot, wlT)


def _segsum_body(table_hbm,
                 src0, dst0, src1, dst1, src2, dst2, src3, dst3, src4, dst4,
                 out_hbm,
                 src_v, dst_v, bufs, accum, gsems, ssems, sem):
    cid = lax.axis_index("c")
    sid = lax.axis_index("s")
    wid = cid * NS + sid
    row0 = sid * ROWS_PER_TILE
    edge_refs = [(src0, dst0), (src1, dst1), (src2, dst2), (src3, dst3),
                 (src4, dst4)]

    def gather(r, c, b):
        pltpu.async_copy(table_hbm.at[r].at[src_v.at[c]], bufs.at[b],
                         gsems.at[b])

    def wait_gather(b):
        pltpu.make_async_copy(table_hbm.at[0].at[src_v.at[0]], bufs.at[b],
                              gsems.at[b]).wait()

    def scatter(c, b):
        pltpu.async_copy(bufs.at[b], accum.at[dst_v.at[c]], ssems.at[b],
                         add=True)

    def wait_scatter(b):
        pltpu.make_async_copy(bufs.at[b], accum.at[dst_v.at[0]],
                              ssems.at[b]).wait()

    zv = jnp.zeros((16,), _f32)

    def zrow(i, carry):
        for j in range(W // 16):
            bufs[0, i, pl.ds(j * 16, 16)] = zv
        return carry

    def zero_slice():
        lax.fori_loop(0, CHUNK, zrow, 0)

        def zcopy(k, carry):
            pltpu.sync_copy(bufs.at[0],
                            accum.at[pl.ds(row0 + k * CHUNK, CHUNK)])
            return carry
        lax.fori_loop(0, ROWS_PER_TILE // CHUNK, zcopy, 0)

    zero_slice()

    for r in range(NREL):
        plsc.subcore_barrier()
        src_hbm, dst_hbm = edge_refs[r]
        for q in range(NQ):
            pltpu.sync_copy(src_hbm.at[wid, pl.ds(q * QCH, QCH)], src_v)
            pltpu.sync_copy(dst_hbm.at[wid, pl.ds(q * QCH, QCH)], dst_v)

            gather(r, 0, 0)
            gather(r, 1, 1)

            def step(p, carry):
                c0 = p * 3
                wait_gather(0)
                gather(r, c0 + 2, 2)
                pltpu.sync_copy(bufs.at[0], accum.at[dst_v.at[c0]],
                                add=True)
                wait_gather(1)
                gather(r, c0 + 3, 0)
                pltpu.sync_copy(bufs.at[1], accum.at[dst_v.at[c0 + 1]],
                                add=True)
                wait_gather(2)

                @pl.when(c0 + 4 < QCH)
                def _():
                    gather(r, c0 + 4, 1)

                pltpu.sync_copy(bufs.at[2], accum.at[dst_v.at[c0 + 2]],
                                add=True)
                return carry

            lax.fori_loop(0, QCH // 3, step, 0)
            wait_gather(0)
            pltpu.sync_copy(bufs.at[0], accum.at[dst_v.at[QCH - 1]],
                            add=True)
        plsc.subcore_barrier()

        pltpu.sync_copy(accum.at[pl.ds(row0, ROWS_PER_TILE)],
                        out_hbm.at[r, cid, pl.ds(row0, ROWS_PER_TILE)])
        if r < NREL - 1:
            zero_slice()


_segsum = functools.partial(
    pl.kernel,
    out_type=jax.ShapeDtypeStruct((NREL, NC, NPAD, W), _f32),
    mesh=plsc.VectorSubcoreMesh(core_axis_name="c", subcore_axis_name="s"),
    scratch_types=[
        pltpu.VMEM((QCH, CHUNK), jnp.int32),
        pltpu.VMEM((QCH, CHUNK), jnp.int32),
        pltpu.VMEM((3, CHUNK, W), _f32),
        pltpu.VMEM_SHARED((NPAD, W), _f32),
        pltpu.SemaphoreType.DMA((4,)),
        pltpu.SemaphoreType.DMA((4,)),
        pltpu.SemaphoreType.DMA,
    ],
    compiler_params=pltpu.CompilerParams(use_tc_tiling_on_sc=False),
)(_segsum_body)


RB3 = 1000


def _combine_body(p_ref, xj_ref, wr_ref, bl_ref, o_ref):
    acc = jnp.dot(xj_ref[...], wr_ref[...], preferred_element_type=_f32)
    acc = acc + bl_ref[...]
    for r in range(NREL):
        s = p_ref[r, 0, :, pl.ds(0, D)] + p_ref[r, 1, :, pl.ds(0, D)]
        c = p_ref[r, 0, :, pl.ds(D, 1)] + p_ref[r, 1, :, pl.ds(D, 1)]
        acc = acc + s / jnp.maximum(c, 1.0)
    o_ref[...] = jnp.maximum(acc, 0.0)


def _combine(partial, x_job, wrT, bl):
    return pl.pallas_call(
        _combine_body,
        grid=(N // RB3,),
        in_specs=[
            pl.BlockSpec((NREL, NC, RB3, W), lambda i: (0, 0, i, 0)),
            pl.BlockSpec((RB3, D), lambda i: (i, 0)),
            pl.BlockSpec((D, D), lambda i: (0, 0)),
            pl.BlockSpec((1, D), lambda i: (0, 0)),
        ],
        out_specs=pl.BlockSpec((RB3, D), lambda i: (i, 0)),
        out_shape=jax.ShapeDtypeStruct((N, D), _f32),
    )(partial, x_job, wrT, bl)


def _prep_edges(ei):
    ei = ei.astype(jnp.int32)
    return ei[0].reshape(NW, NCH, CHUNK), ei[1].reshape(NW, NCH, CHUNK)


def kernel(x_station, x_machine, x_robot, x_job,
           edge_can_load, edge_loaded, edge_will_execute, edge_execute,
           edge_hold,
           Wl_can_load, bl_can_load, Wr_can_load,
           Wl_loaded, bl_loaded, Wr_loaded,
           Wl_will_execute, bl_will_execute, Wr_will_execute,
           Wl_execute, bl_execute, Wr_execute,
           Wl_hold, bl_hold, Wr_hold):
    wlT = jnp.stack([Wl_can_load.T, Wl_loaded.T, Wl_will_execute.T,
                     Wl_execute.T, Wl_hold.T])
    table = _build_tables(x_station, x_machine, x_robot, wlT)

    pairs = [_prep_edges(e) for e in (edge_can_load, edge_loaded,
                                      edge_will_execute, edge_execute,
                                      edge_hold)]
    flat = [a for p in pairs for a in p]
    partial = _segsum(table, *flat)

    wrT = (Wr_can_load + Wr_loaded + Wr_will_execute + Wr_execute + Wr_hold).T
    bl = (bl_can_load + bl_loaded + bl_will_execute + bl_execute
          + bl_hold).reshape(1, D)
    return _combine(partial, x_job, wrT, bl)


# speedup vs baseline: 1.7734x; 1.0560x over previous
import functools

import jax
import jax.numpy as jnp
from jax import lax
from jax.experimental import pallas as pl
from jax.experimental.pallas import tpu as pltpu
from jax.experimental.pallas import tpu_sc as plsc

N = 10000
D = 128
E = 320000
NREL = 5
W = 144
NPAD = 10240
NC, NS = 2, 16
NW = NC * NS
ROWS_PER_TILE = NPAD // NS
HALF = ROWS_PER_TILE // 2
NCH = 125
CHUNK = 80
EPT = NCH * CHUNK
NQ = 5
QCH = NCH // NQ

_f32 = jnp.float32


RB1 = 1280


def _table_body(xs_ref, xm_ref, xr_ref, w_ref, o_ref):
    tail = jnp.concatenate(
        [jnp.ones((RB1, 1), _f32), jnp.zeros((RB1, W - D - 1), _f32)], axis=1)
    srcs = (xs_ref, xs_ref, xm_ref, xm_ref, xr_ref)
    for r in range(NREL):
        y = jnp.dot(srcs[r][...], w_ref[r], preferred_element_type=_f32)
        o_ref[r, :, pl.ds(0, D)] = y
        o_ref[r, :, pl.ds(D, W - D)] = tail


def _build_tables(x_station, x_machine, x_robot, wlT):
    xspec = pl.BlockSpec((RB1, D), lambda i: (i, 0))
    return pl.pallas_call(
        _table_body,
        grid=(NPAD // RB1,),
        in_specs=[
            xspec, xspec, xspec,
            pl.BlockSpec((NREL, D, D), lambda i: (0, 0, 0)),
        ],
        out_specs=pl.BlockSpec((NREL, RB1, W), lambda i: (0, i, 0)),
        out_shape=jax.ShapeDtypeStruct((NREL, NPAD, W), _f32),
    )(x_station, x_machine, x_robot, wlT)


def _segsum_body(table_hbm, e0, e1, e2, e3, e4, out_hbm,
                 src_v, dst_v, bufs, accum, gsems, ssems, sem):
    cid = lax.axis_index("c")
    sid = lax.axis_index("s")
    wid = cid * NS + sid
    row0 = sid * ROWS_PER_TILE
    edge_refs = [e0, e1, e2, e3, e4]

    def gather(r, c, b):
        pltpu.async_copy(
            table_hbm.at[r].at[src_v.at[pl.ds(c * CHUNK, CHUNK)]],
            bufs.at[b], gsems.at[b])

    def wait_gather(b):
        pltpu.make_async_copy(
            table_hbm.at[0].at[src_v.at[pl.ds(0, CHUNK)]],
            bufs.at[b], gsems.at[b]).wait()

    def scatter(c, b):
        pltpu.async_copy(bufs.at[b],
                         accum.at[dst_v.at[pl.ds(c * CHUNK, CHUNK)]],
                         ssems.at[b], add=True)

    zv = jnp.zeros((16,), _f32)

    def zrow(i, carry):
        for j in range(W // 16):
            bufs[0, i, pl.ds(j * 16, 16)] = zv
        return carry

    def zero_slice():
        lax.fori_loop(0, CHUNK, zrow, 0)

        def zcopy(k, carry):
            pltpu.sync_copy(bufs.at[0],
                            accum.at[pl.ds(row0 + k * CHUNK, CHUNK)])
            return carry
        lax.fori_loop(0, ROWS_PER_TILE // CHUNK, zcopy, 0)

    zero_slice()

    def scatter_sync(c, b):
        pltpu.sync_copy(bufs.at[b],
                        accum.at[dst_v.at[pl.ds(c * CHUNK, CHUNK)]],
                        add=True)

    for r in range(NREL):
        plsc.subcore_barrier()
        e_hbm = edge_refs[r]
        for q in range(NQ):
            base = wid * EPT + q * (QCH * CHUNK)
            pltpu.sync_copy(e_hbm.at[0, pl.ds(base, QCH * CHUNK)], src_v)
            pltpu.sync_copy(e_hbm.at[1, pl.ds(base, QCH * CHUNK)], dst_v)

            gather(r, 0, 0)
            gather(r, 1, 1)

            def step(p, carry):
                c0 = p * 3
                wait_gather(0)
                gather(r, c0 + 2, 2)
                scatter_sync(c0, 0)
                wait_gather(1)
                gather(r, c0 + 3, 0)
                scatter_sync(c0 + 1, 1)
                wait_gather(2)

                @pl.when(c0 + 4 < QCH)
                def _():
                    gather(r, c0 + 4, 1)

                scatter_sync(c0 + 2, 2)
                return carry

            lax.fori_loop(0, QCH // 3, step, 0)
            wait_gather(0)
            scatter_sync(QCH - 1, 0)
        plsc.subcore_barrier()

        pltpu.sync_copy(accum.at[pl.ds(row0, ROWS_PER_TILE)],
                        out_hbm.at[r, cid, pl.ds(row0, ROWS_PER_TILE)])
        if r < NREL - 1:
            zero_slice()


_segsum = functools.partial(
    pl.kernel,
    out_type=jax.ShapeDtypeStruct((NREL, NC, NPAD, W), _f32),
    mesh=plsc.VectorSubcoreMesh(core_axis_name="c", subcore_axis_name="s"),
    scratch_types=[
        pltpu.VMEM((QCH * CHUNK,), jnp.int32),
        pltpu.VMEM((QCH * CHUNK,), jnp.int32),
        pltpu.VMEM((3, CHUNK, W), _f32),
        pltpu.VMEM_SHARED((NPAD, W), _f32),
        pltpu.SemaphoreType.DMA((4,)),
        pltpu.SemaphoreType.DMA((4,)),
        pltpu.SemaphoreType.DMA,
    ],
    compiler_params=pltpu.CompilerParams(use_tc_tiling_on_sc=False),
)(_segsum_body)


RB3 = 1000


def _combine_body(p_ref, xj_ref, wr_ref, bl_ref, o_ref):
    acc = jnp.dot(xj_ref[...], wr_ref[...], preferred_element_type=_f32)
    acc = acc + bl_ref[...]
    for r in range(NREL):
        s = p_ref[r, 0, :, pl.ds(0, D)] + p_ref[r, 1, :, pl.ds(0, D)]
        c = p_ref[r, 0, :, pl.ds(D, 1)] + p_ref[r, 1, :, pl.ds(D, 1)]
        acc = acc + s / jnp.maximum(c, 1.0)
    o_ref[...] = jnp.maximum(acc, 0.0)


def _combine(partial, x_job, wrT, bl):
    return pl.pallas_call(
        _combine_body,
        grid=(N // RB3,),
        in_specs=[
            pl.BlockSpec((NREL, NC, RB3, W), lambda i: (0, 0, i, 0)),
            pl.BlockSpec((RB3, D), lambda i: (i, 0)),
            pl.BlockSpec((D, D), lambda i: (0, 0)),
            pl.BlockSpec((1, D), lambda i: (0, 0)),
        ],
        out_specs=pl.BlockSpec((RB3, D), lambda i: (i, 0)),
        out_shape=jax.ShapeDtypeStruct((N, D), _f32),
    )(partial, x_job, wrT, bl)


def kernel(x_station, x_machine, x_robot, x_job,
           edge_can_load, edge_loaded, edge_will_execute, edge_execute,
           edge_hold,
           Wl_can_load, bl_can_load, Wr_can_load,
           Wl_loaded, bl_loaded, Wr_loaded,
           Wl_will_execute, bl_will_execute, Wr_will_execute,
           Wl_execute, bl_execute, Wr_execute,
           Wl_hold, bl_hold, Wr_hold):
    wlT = jnp.stack([Wl_can_load.T, Wl_loaded.T, Wl_will_execute.T,
                     Wl_execute.T, Wl_hold.T])
    table = _build_tables(x_station, x_machine, x_robot, wlT)

    edges = [e.astype(jnp.int32) for e in (edge_can_load, edge_loaded,
                                           edge_will_execute, edge_execute,
                                           edge_hold)]
    partial = _segsum(table, *edges)

    wrT = (Wr_can_load + Wr_loaded + Wr_will_execute + Wr_execute + Wr_hold).T
    bl = (bl_can_load + bl_loaded + bl_will_execute + bl_execute
          + bl_hold).reshape(1, D)
    return _combine(partial, x_job, wrT, bl)


# speedup vs baseline: 1.8028x; 1.0166x over previous
import functools

import jax
import jax.numpy as jnp
from jax import lax
from jax.experimental import pallas as pl
from jax.experimental.pallas import tpu as pltpu
from jax.experimental.pallas import tpu_sc as plsc

N = 10000
D = 128
E = 320000
NREL = 5
W = 144
NPAD = 10240
NC, NS = 2, 16
NW = NC * NS
ROWS_PER_TILE = NPAD // NS
HALF = ROWS_PER_TILE // 2
NCH = 125
CHUNK = 80
EPT = NCH * CHUNK
NQ = 5
QCH = NCH // NQ

_f32 = jnp.float32


RB1 = 1280


def _table_body(xs_ref, xm_ref, xr_ref, w_ref, o_ref):
    tail = jnp.concatenate(
        [jnp.ones((RB1, 1), _f32), jnp.zeros((RB1, W - D - 1), _f32)], axis=1)
    srcs = (xs_ref, xs_ref, xm_ref, xm_ref, xr_ref)
    for r in range(NREL):
        y = jnp.dot(srcs[r][...], w_ref[r], preferred_element_type=_f32)
        o_ref[r, :, pl.ds(0, D)] = y
        o_ref[r, :, pl.ds(D, W - D)] = tail


def _build_tables(x_station, x_machine, x_robot, wlT):
    xspec = pl.BlockSpec((RB1, D), lambda i: (i, 0))
    return pl.pallas_call(
        _table_body,
        grid=(NPAD // RB1,),
        in_specs=[
            xspec, xspec, xspec,
            pl.BlockSpec((NREL, D, D), lambda i: (0, 0, 0)),
        ],
        out_specs=pl.BlockSpec((NREL, RB1, W), lambda i: (0, i, 0)),
        out_shape=jax.ShapeDtypeStruct((NREL, NPAD, W), _f32),
    )(x_station, x_machine, x_robot, wlT)


def _segsum_body(table_hbm, e0, e1, e2, e3, e4, out_hbm,
                 src_v, dst_v, bufs, accum, gsems, ssems, sem):
    cid = lax.axis_index("c")
    sid = lax.axis_index("s")
    wid = cid * NS + sid
    row0 = sid * ROWS_PER_TILE
    edge_refs = [e0, e1, e2, e3, e4]

    def gather(r, c, b):
        pltpu.async_copy(
            table_hbm.at[r].at[src_v.at[pl.ds(c * CHUNK, CHUNK)]],
            bufs.at[b], gsems.at[b])

    def wait_gather(b):
        pltpu.make_async_copy(
            table_hbm.at[0].at[src_v.at[pl.ds(0, CHUNK)]],
            bufs.at[b], gsems.at[b]).wait()

    def scatter(c, b):
        pltpu.async_copy(bufs.at[b],
                         accum.at[dst_v.at[pl.ds(c * CHUNK, CHUNK)]],
                         ssems.at[b], add=True)

    zv = jnp.zeros((16,), _f32)

    def zrow(i, carry):
        for j in range(W // 16):
            bufs[0, i, pl.ds(j * 16, 16)] = zv
        return carry

    def zero_slice():
        lax.fori_loop(0, CHUNK, zrow, 0)

        def zcopy(k, carry):
            pltpu.sync_copy(bufs.at[0],
                            accum.at[pl.ds(row0 + k * CHUNK, CHUNK)])
            return carry
        lax.fori_loop(0, ROWS_PER_TILE // CHUNK, zcopy, 0)

    zero_slice()

    def scatter_sync(c, b):
        pltpu.sync_copy(bufs.at[b],
                        accum.at[dst_v.at[pl.ds(c * CHUNK, CHUNK)]],
                        add=True)

    for r in range(NREL):
        plsc.subcore_barrier()
        e_hbm = edge_refs[r]
        for q in range(NQ):
            base = wid * EPT + q * (QCH * CHUNK)
            s_cp = pltpu.async_copy(e_hbm.at[0, pl.ds(base, QCH * CHUNK)],
                                    src_v, ssems.at[0])
            d_cp = pltpu.async_copy(e_hbm.at[1, pl.ds(base, QCH * CHUNK)],
                                    dst_v, ssems.at[1])
            s_cp.wait()
            gather(r, 0, 0)
            gather(r, 1, 1)
            d_cp.wait()

            def step(p, carry):
                c0 = p * 3
                wait_gather(0)
                gather(r, c0 + 2, 2)
                scatter_sync(c0, 0)
                wait_gather(1)
                gather(r, c0 + 3, 0)
                scatter_sync(c0 + 1, 1)
                wait_gather(2)

                @pl.when(c0 + 4 < QCH)
                def _():
                    gather(r, c0 + 4, 1)

                scatter_sync(c0 + 2, 2)
                return carry

            lax.fori_loop(0, QCH // 3, step, 0)
            wait_gather(0)
            scatter_sync(QCH - 1, 0)
        plsc.subcore_barrier()

        pltpu.sync_copy(accum.at[pl.ds(row0, ROWS_PER_TILE)],
                        out_hbm.at[r, cid, pl.ds(row0, ROWS_PER_TILE)])
        if r < NREL - 1:
            zero_slice()


_segsum = functools.partial(
    pl.kernel,
    out_type=jax.ShapeDtypeStruct((NREL, NC, NPAD, W), _f32),
    mesh=plsc.VectorSubcoreMesh(core_axis_name="c", subcore_axis_name="s"),
    scratch_types=[
        pltpu.VMEM((QCH * CHUNK,), jnp.int32),
        pltpu.VMEM((QCH * CHUNK,), jnp.int32),
        pltpu.VMEM((3, CHUNK, W), _f32),
        pltpu.VMEM_SHARED((NPAD, W), _f32),
        pltpu.SemaphoreType.DMA((4,)),
        pltpu.SemaphoreType.DMA((4,)),
        pltpu.SemaphoreType.DMA,
    ],
    compiler_params=pltpu.CompilerParams(use_tc_tiling_on_sc=False),
)(_segsum_body)


RB3 = 2000


def _combine_body(p_ref, xj_ref, wr_ref, bl_ref, o_ref):
    acc = jnp.dot(xj_ref[...], wr_ref[...], preferred_element_type=_f32)
    acc = acc + bl_ref[...]
    for r in range(NREL):
        s = p_ref[r, 0, :, pl.ds(0, D)] + p_ref[r, 1, :, pl.ds(0, D)]
        c = p_ref[r, 0, :, pl.ds(D, 1)] + p_ref[r, 1, :, pl.ds(D, 1)]
        acc = acc + s / jnp.maximum(c, 1.0)
    o_ref[...] = jnp.maximum(acc, 0.0)


def _combine(partial, x_job, wrT, bl):
    return pl.pallas_call(
        _combine_body,
        grid=(N // RB3,),
        in_specs=[
            pl.BlockSpec((NREL, NC, RB3, W), lambda i: (0, 0, i, 0)),
            pl.BlockSpec((RB3, D), lambda i: (i, 0)),
            pl.BlockSpec((D, D), lambda i: (0, 0)),
            pl.BlockSpec((1, D), lambda i: (0, 0)),
        ],
        out_specs=pl.BlockSpec((RB3, D), lambda i: (i, 0)),
        out_shape=jax.ShapeDtypeStruct((N, D), _f32),
    )(partial, x_job, wrT, bl)


def kernel(x_station, x_machine, x_robot, x_job,
           edge_can_load, edge_loaded, edge_will_execute, edge_execute,
           edge_hold,
           Wl_can_load, bl_can_load, Wr_can_load,
           Wl_loaded, bl_loaded, Wr_loaded,
           Wl_will_execute, bl_will_execute, Wr_will_execute,
           Wl_execute, bl_execute, Wr_execute,
           Wl_hold, bl_hold, Wr_hold):
    wlT = jnp.stack([Wl_can_load.T, Wl_loaded.T, Wl_will_execute.T,
                     Wl_execute.T, Wl_hold.T])
    table = _build_tables(x_station, x_machine, x_robot, wlT)

    edges = [e.astype(jnp.int32) for e in (edge_can_load, edge_loaded,
                                           edge_will_execute, edge_execute,
                                           edge_hold)]
    partial = _segsum(table, *edges)

    wrT = (Wr_can_load + Wr_loaded + Wr_will_execute + Wr_execute + Wr_hold).T
    bl = (bl_can_load + bl_loaded + bl_will_execute + bl_execute
          + bl_hold).reshape(1, D)
    return _combine(partial, x_job, wrT, bl)
